# baseline (speedup 1.0000x reference)
import functools
import math

import jax
import jax.numpy as jnp
from jax import lax
from jax.experimental import pallas as pl
from jax.experimental.pallas import tpu as pltpu

_BLK = 1024


def _w_pattern_body(m_ref, u_ref, o_ref, acc_ref, *, nr):
    r = pl.program_id(2)

    @pl.when(r == 0)
    def _init():
        acc_ref[...] = jnp.zeros_like(acc_ref)

    acc_ref[...] += jnp.dot(m_ref[...], u_ref[...],
                            preferred_element_type=jnp.float32)

    @pl.when(r == nr - 1)
    def _done():
        o_ref[...] = (acc_ref[...] > 0.0).astype(jnp.bfloat16)


def _w_pattern(m, u):
    np_, kp = m.shape[0], u.shape[1]
    b = _BLK
    ni, nj, nr = np_ // b, kp // b, np_ // b
    return pl.pallas_call(
        functools.partial(_w_pattern_body, nr=nr),
        grid=(ni, nj, nr),
        in_specs=[
            pl.BlockSpec((b, b), lambda i, j, r: (i, r)),
            pl.BlockSpec((b, b), lambda i, j, r: (r, j)),
        ],
        out_specs=pl.BlockSpec((b, b), lambda i, j, r: (i, j)),
        out_shape=jax.ShapeDtypeStruct((np_, kp), jnp.bfloat16),
        scratch_shapes=[pltpu.VMEM((b, b), jnp.float32)],
    )(m, u)


def _mask_stage_body(ut_ref, w_ref, xa_ref, o_ref, acc_ref, *, nr, b):
    j = pl.program_id(0)
    i = pl.program_id(1)
    r = pl.program_id(2)

    @pl.when(r == 0)
    def _init():
        acc_ref[...] = jnp.zeros_like(acc_ref)

    acc_ref[...] += jnp.dot(ut_ref[...], w_ref[...],
                            preferred_element_type=jnp.float32)

    @pl.when(r == nr - 1)
    def _done():
        gi = i * b + lax.broadcasted_iota(jnp.int32, (b, b), 0)
        gj = j * b + lax.broadcasted_iota(jnp.int32, (b, b), 1)
        mask = jnp.where((acc_ref[...] > 0.0) & (gi != gj), 1.0, 0.0)
        contrib = lax.dot_general(mask, xa_ref[...],
                                  (((0,), (0,)), ((), ())),
                                  preferred_element_type=jnp.float32)

        @pl.when(i == 0)
        def _set():
            o_ref[...] = contrib

        @pl.when(i != 0)
        def _add():
            o_ref[...] += contrib


def _mask_stage(ut, w, xa):
    kp, np_ = ut.shape
    f = xa.shape[1]
    b = _BLK
    nj, ni, nr = kp // b, kp // b, np_ // b
    return pl.pallas_call(
        functools.partial(_mask_stage_body, nr=nr, b=b),
        grid=(nj, ni, nr),
        in_specs=[
            pl.BlockSpec((b, b), lambda j, i, r: (i, r)),
            pl.BlockSpec((b, b), lambda j, i, r: (r, j)),
            pl.BlockSpec((b, f), lambda j, i, r: (i, 0)),
        ],
        out_specs=pl.BlockSpec((b, f), lambda j, i, r: (j, 0)),
        out_shape=jax.ShapeDtypeStruct((kp, f), jnp.float32),
        scratch_shapes=[pltpu.VMEM((b, b), jnp.float32)],
    )(ut, w, xa)


def _final_reduce_body(wd_ref, xa_ref, o_ref, *, d):
    q = pl.program_id(0)

    @pl.when(q == 0)
    def _init():
        o_ref[...] = jnp.zeros_like(o_ref)

    w = wd_ref[:, :d]
    deg = wd_ref[:, d:d + 1]
    m = w / jnp.maximum(deg, 1.0)
    o_ref[0:1, :d] += jnp.sum(m, axis=0, keepdims=True)
    o_ref[1:2, :d] += jnp.sum(xa_ref[:, :d], axis=0, keepdims=True)


def _final_reduce(wd, xa, d):
    kp, f = wd.shape
    b = _BLK
    return pl.pallas_call(
        functools.partial(_final_reduce_body, d=d),
        grid=(kp // b,),
        in_specs=[
            pl.BlockSpec((b, f), lambda q: (q, 0)),
            pl.BlockSpec((b, f), lambda q: (q, 0)),
        ],
        out_specs=pl.BlockSpec((8, f), lambda q: (0, 0)),
        out_shape=jax.ShapeDtypeStruct((8, f), jnp.float32),
    )(wd, xa)


def kernel(x, edge_index, batch, W_l1, b_l1, W_r1, W_lin, b_lin, W_att, b_att,
           W_le1, b_le1, W_le2, W_le3, b_le3, W_l2, b_l2, W_r2):
    n, d = x.shape
    k = int(math.ceil(0.5 * n))
    src = edge_index[0]
    dst = edge_index[1]
    ones_e = jnp.ones(src.shape, jnp.float32)

    msum = jax.ops.segment_sum(x[src], dst, num_segments=n)
    cnt = jax.ops.segment_sum(ones_e, dst, num_segments=n)
    mean1 = msum / jnp.clip(cnt, 1.0, None)[:, None]
    h = jax.nn.relu(mean1 @ W_l1.T + b_l1 + x @ W_r1.T)

    x_q = jnp.maximum(jax.ops.segment_max(h[src], dst, num_segments=n), h)
    wq = W_att[0, :d]
    wj = W_att[0, d:]
    alpha = x_q @ (W_lin.T @ wq) + (b_lin @ wq + b_att[0])
    beta = h @ wj
    s_loop = jax.nn.leaky_relu(alpha + beta, 0.2)
    s_e = jax.nn.leaky_relu(alpha[dst] + beta[src], 0.2)
    smax = jnp.maximum(jax.ops.segment_max(s_e, dst, num_segments=n), s_loop)
    ex_e = jnp.exp(s_e - smax[dst])
    ex_loop = jnp.exp(s_loop - smax)
    den = jax.ops.segment_sum(ex_e, dst, num_segments=n) + ex_loop
    sc_e = ex_e / den[dst]
    sc_loop = ex_loop / den
    xp = (jax.ops.segment_sum(h[src] * sc_e[:, None], dst, num_segments=n)
          + h * sc_loop[:, None])

    a_n = xp @ W_le1[0] + b_le1[0]
    b_n = xp @ W_le2[0]
    agg = (jax.ops.segment_sum(a_n[src], dst, num_segments=n) + a_n
           - (cnt + 1.0) * b_n)
    fitness = jax.nn.sigmoid(agg + xp @ W_le3[0] + b_le3[0])

    _, idx = lax.top_k(fitness, k)
    sel = jnp.sort(idx)
    xsel = xp[sel] * fitness[sel][:, None]

    b_ = _BLK
    np_ = ((n + b_ - 1) // b_) * b_
    kp = ((k + b_ - 1) // b_) * b_
    loops = jnp.arange(n, dtype=src.dtype)
    row = jnp.concatenate([src, loops])
    col = jnp.concatenate([dst, loops])
    one_b = jnp.ones((), jnp.bfloat16)
    m_pat = jnp.zeros((np_, np_), jnp.bfloat16).at[row, col].set(one_b)
    pos = jnp.searchsorted(sel, col).astype(jnp.int32)
    hit = sel[jnp.clip(pos, 0, k - 1)] == col
    posd = jnp.where(hit, pos, kp + 1)
    u_pat = jnp.zeros((np_, kp), jnp.bfloat16).at[row, posd].set(
        one_b, mode="drop")
    ut_pat = jnp.zeros((kp, np_), jnp.bfloat16).at[posd, row].set(
        one_b, mode="drop")

    w_pat = _w_pattern(m_pat, u_pat)

    f = 2 * d
    xa = jnp.zeros((kp, f), jnp.float32)
    xa = xa.at[:k, :d].set(xsel)
    xa = xa.at[:k, d].set(1.0)
    wd = _mask_stage(ut_pat, w_pat, xa)
    red = _final_reduce(wd, xa, d)
    mean_sum = red[0, :d]
    xsel_sum = red[1, :d]
    out = (mean_sum @ W_l2.T + k * b_l2 + xsel_sum @ W_r2.T) / k
    return out.reshape(1, d)


# speedup vs baseline: 1.1680x; 1.1680x over previous
import functools
import math

import jax
import jax.numpy as jnp
from jax import lax
from jax.experimental import pallas as pl
from jax.experimental.pallas import tpu as pltpu

_BLK = 1024


def _w_pattern_body(m_ref, u_ref, o_ref, acc_ref, *, nr):
    r = pl.program_id(2)

    @pl.when(r == 0)
    def _init():
        acc_ref[...] = jnp.zeros_like(acc_ref)

    acc_ref[...] += jnp.dot(m_ref[...], u_ref[...],
                            preferred_element_type=jnp.float32)

    @pl.when(r == nr - 1)
    def _done():
        o_ref[...] = (acc_ref[...] > 0.0).astype(jnp.bfloat16)


def _w_pattern(m, u):
    np_, kp = m.shape[0], u.shape[1]
    b = _BLK
    ni, nj, nr = np_ // b, kp // b, np_ // b
    return pl.pallas_call(
        functools.partial(_w_pattern_body, nr=nr),
        grid=(ni, nj, nr),
        in_specs=[
            pl.BlockSpec((b, b), lambda i, j, r: (i, r)),
            pl.BlockSpec((b, b), lambda i, j, r: (r, j)),
        ],
        out_specs=pl.BlockSpec((b, b), lambda i, j, r: (i, j)),
        out_shape=jax.ShapeDtypeStruct((np_, kp), jnp.bfloat16),
        scratch_shapes=[pltpu.VMEM((b, b), jnp.float32)],
    )(m, u)


def _mask_stage_body(ut_ref, w_ref, xa_ref, o_ref, acc_ref, *, nr, b):
    j = pl.program_id(0)
    i = pl.program_id(1)
    r = pl.program_id(2)

    @pl.when(r == 0)
    def _init():
        acc_ref[...] = jnp.zeros_like(acc_ref)

    acc_ref[...] += jnp.dot(ut_ref[...], w_ref[...],
                            preferred_element_type=jnp.float32)

    @pl.when(r == nr - 1)
    def _done():
        gi = i * b + lax.broadcasted_iota(jnp.int32, (b, b), 0)
        gj = j * b + lax.broadcasted_iota(jnp.int32, (b, b), 1)
        mask = jnp.where((acc_ref[...] > 0.0) & (gi != gj), 1.0, 0.0)
        contrib = lax.dot_general(mask, xa_ref[...],
                                  (((0,), (0,)), ((), ())),
                                  preferred_element_type=jnp.float32)

        @pl.when(i == 0)
        def _set():
            o_ref[...] = contrib

        @pl.when(i != 0)
        def _add():
            o_ref[...] += contrib


def _mask_stage(ut, w, xa):
    kp, np_ = ut.shape
    f = xa.shape[1]
    b = _BLK
    nj, ni, nr = kp // b, kp // b, np_ // b
    return pl.pallas_call(
        functools.partial(_mask_stage_body, nr=nr, b=b),
        grid=(nj, ni, nr),
        in_specs=[
            pl.BlockSpec((b, b), lambda j, i, r: (i, r)),
            pl.BlockSpec((b, b), lambda j, i, r: (r, j)),
            pl.BlockSpec((b, f), lambda j, i, r: (i, 0)),
        ],
        out_specs=pl.BlockSpec((b, f), lambda j, i, r: (j, 0)),
        out_shape=jax.ShapeDtypeStruct((kp, f), jnp.float32),
        scratch_shapes=[pltpu.VMEM((b, b), jnp.float32)],
    )(ut, w, xa)


def _final_reduce_body(wd_ref, xa_ref, o_ref, *, d):
    q = pl.program_id(0)

    @pl.when(q == 0)
    def _init():
        o_ref[...] = jnp.zeros_like(o_ref)

    w = wd_ref[:, :d]
    deg = wd_ref[:, d:d + 1]
    m = w / jnp.maximum(deg, 1.0)
    o_ref[0:1, :d] += jnp.sum(m, axis=0, keepdims=True)
    o_ref[1:2, :d] += jnp.sum(xa_ref[:, :d], axis=0, keepdims=True)


def _final_reduce(wd, xa, d):
    kp, f = wd.shape
    b = _BLK
    return pl.pallas_call(
        functools.partial(_final_reduce_body, d=d),
        grid=(kp // b,),
        in_specs=[
            pl.BlockSpec((b, f), lambda q: (q, 0)),
            pl.BlockSpec((b, f), lambda q: (q, 0)),
        ],
        out_specs=pl.BlockSpec((8, f), lambda q: (0, 0)),
        out_shape=jax.ShapeDtypeStruct((8, f), jnp.float32),
    )(wd, xa)


def kernel(x, edge_index, batch, W_l1, b_l1, W_r1, W_lin, b_lin, W_att, b_att,
           W_le1, b_le1, W_le2, W_le3, b_le3, W_l2, b_l2, W_r2):
    n, d = x.shape
    k = int(math.ceil(0.5 * n))
    src = edge_index[0]
    dst = edge_index[1]
    ones_e = jnp.ones(src.shape, jnp.float32)

    msum = jax.ops.segment_sum(x[src], dst, num_segments=n)
    cnt = jax.ops.segment_sum(ones_e, dst, num_segments=n)
    mean1 = msum / jnp.clip(cnt, 1.0, None)[:, None]
    h = jax.nn.relu(mean1 @ W_l1.T + b_l1 + x @ W_r1.T)

    x_q = jnp.maximum(jax.ops.segment_max(h[src], dst, num_segments=n), h)
    wq = W_att[0, :d]
    wj = W_att[0, d:]
    alpha = x_q @ (W_lin.T @ wq) + (b_lin @ wq + b_att[0])
    beta = h @ wj
    s_loop = jax.nn.leaky_relu(alpha + beta, 0.2)
    s_e = jax.nn.leaky_relu(alpha[dst] + beta[src], 0.2)
    smax = jnp.maximum(jax.ops.segment_max(s_e, dst, num_segments=n), s_loop)
    ex_e = jnp.exp(s_e - smax[dst])
    ex_loop = jnp.exp(s_loop - smax)
    den = jax.ops.segment_sum(ex_e, dst, num_segments=n) + ex_loop
    sc_e = ex_e / den[dst]
    sc_loop = ex_loop / den
    xp = (jax.ops.segment_sum(h[src] * sc_e[:, None], dst, num_segments=n)
          + h * sc_loop[:, None])

    a_n = xp @ W_le1[0] + b_le1[0]
    b_n = xp @ W_le2[0]
    agg = (jax.ops.segment_sum(a_n[src], dst, num_segments=n) + a_n
           - (cnt + 1.0) * b_n)
    fitness = jax.nn.sigmoid(agg + xp @ W_le3[0] + b_le3[0])

    _, idx = lax.top_k(fitness, k)
    sel = jnp.sort(idx)
    xsel = xp[sel] * fitness[sel][:, None]

    b_ = _BLK
    np_ = ((n + b_ - 1) // b_) * b_
    kp = ((k + b_ - 1) // b_) * b_
    loops = jnp.arange(n, dtype=src.dtype)
    row = jnp.concatenate([src, loops])
    col = jnp.concatenate([dst, loops])
    ones_t = jnp.ones(row.shape, jnp.float32)
    m_pat = jnp.zeros((np_, np_), jnp.float32).at[row, col].add(
        ones_t).astype(jnp.bfloat16)
    pos = jnp.searchsorted(sel, col).astype(jnp.int32)
    hit = sel[jnp.clip(pos, 0, k - 1)] == col
    posd = jnp.where(hit, pos, kp + 1)
    u_pat = jnp.zeros((np_, kp), jnp.float32).at[row, posd].add(
        ones_t, mode="drop").astype(jnp.bfloat16)
    ut_pat = u_pat.T

    w_pat = _w_pattern(m_pat, u_pat)

    f = 2 * d
    xa = jnp.zeros((kp, f), jnp.float32)
    xa = xa.at[:k, :d].set(xsel)
    xa = xa.at[:k, d].set(1.0)
    wd = _mask_stage(ut_pat, w_pat, xa)
    red = _final_reduce(wd, xa, d)
    mean_sum = red[0, :d]
    xsel_sum = red[1, :d]
    out = (mean_sum @ W_l2.T + k * b_l2 + xsel_sum @ W_r2.T) / k
    return out.reshape(1, d)


# speedup vs baseline: 2.2903x; 1.9608x over previous
import functools
import math

import jax
import jax.numpy as jnp
from jax import lax
from jax.experimental import pallas as pl
from jax.experimental.pallas import tpu as pltpu

_BLK = 1024


def _w_pattern_body(m_ref, u_ref, o_ref, acc_ref, *, nr):
    r = pl.program_id(2)

    @pl.when(r == 0)
    def _init():
        acc_ref[...] = jnp.zeros_like(acc_ref)

    acc_ref[...] += jnp.dot(m_ref[...], u_ref[...],
                            preferred_element_type=jnp.float32)

    @pl.when(r == nr - 1)
    def _done():
        o_ref[...] = (acc_ref[...] > 0.0).astype(jnp.bfloat16)


def _w_pattern(m, u):
    np_, kp = m.shape[0], u.shape[1]
    b = _BLK
    ni, nj, nr = np_ // b, kp // b, np_ // b
    return pl.pallas_call(
        functools.partial(_w_pattern_body, nr=nr),
        grid=(ni, nj, nr),
        in_specs=[
            pl.BlockSpec((b, b), lambda i, j, r: (i, r)),
            pl.BlockSpec((b, b), lambda i, j, r: (r, j)),
        ],
        out_specs=pl.BlockSpec((b, b), lambda i, j, r: (i, j)),
        out_shape=jax.ShapeDtypeStruct((np_, kp), jnp.bfloat16),
        scratch_shapes=[pltpu.VMEM((b, b), jnp.float32)],
    )(m, u)


def _mask_stage_body(ut_ref, w_ref, xa_ref, o_ref, acc_ref, *, nr, b):
    j = pl.program_id(0)
    i = pl.program_id(1)
    r = pl.program_id(2)

    @pl.when(r == 0)
    def _init():
        acc_ref[...] = jnp.zeros_like(acc_ref)

    acc_ref[...] += jnp.dot(ut_ref[...], w_ref[...],
                            preferred_element_type=jnp.float32)

    @pl.when(r == nr - 1)
    def _done():
        gi = i * b + lax.broadcasted_iota(jnp.int32, (b, b), 0)
        gj = j * b + lax.broadcasted_iota(jnp.int32, (b, b), 1)
        mask = jnp.where((acc_ref[...] > 0.0) & (gi != gj), 1.0, 0.0)
        contrib = lax.dot_general(mask, xa_ref[...],
                                  (((0,), (0,)), ((), ())),
                                  preferred_element_type=jnp.float32)

        @pl.when(i == 0)
        def _set():
            o_ref[...] = contrib

        @pl.when(i != 0)
        def _add():
            o_ref[...] += contrib


def _mask_stage(ut, w, xa):
    kp, np_ = ut.shape
    f = xa.shape[1]
    b = _BLK
    nj, ni, nr = kp // b, kp // b, np_ // b
    return pl.pallas_call(
        functools.partial(_mask_stage_body, nr=nr, b=b),
        grid=(nj, ni, nr),
        in_specs=[
            pl.BlockSpec((b, b), lambda j, i, r: (i, r)),
            pl.BlockSpec((b, b), lambda j, i, r: (r, j)),
            pl.BlockSpec((b, f), lambda j, i, r: (i, 0)),
        ],
        out_specs=pl.BlockSpec((b, f), lambda j, i, r: (j, 0)),
        out_shape=jax.ShapeDtypeStruct((kp, f), jnp.float32),
        scratch_shapes=[pltpu.VMEM((b, b), jnp.float32)],
    )(ut, w, xa)


def _final_reduce_body(wd_ref, xa_ref, o_ref, *, d):
    q = pl.program_id(0)

    @pl.when(q == 0)
    def _init():
        o_ref[...] = jnp.zeros_like(o_ref)

    w = wd_ref[:, :d]
    deg = wd_ref[:, d:d + 1]
    m = w / jnp.maximum(deg, 1.0)
    o_ref[0:1, :d] += jnp.sum(m, axis=0, keepdims=True)
    o_ref[1:2, :d] += jnp.sum(xa_ref[:, :d], axis=0, keepdims=True)


def _final_reduce(wd, xa, d):
    kp, f = wd.shape
    b = _BLK
    return pl.pallas_call(
        functools.partial(_final_reduce_body, d=d),
        grid=(kp // b,),
        in_specs=[
            pl.BlockSpec((b, f), lambda q: (q, 0)),
            pl.BlockSpec((b, f), lambda q: (q, 0)),
        ],
        out_specs=pl.BlockSpec((8, f), lambda q: (0, 0)),
        out_shape=jax.ShapeDtypeStruct((8, f), jnp.float32),
    )(wd, xa)


def kernel(x, edge_index, batch, W_l1, b_l1, W_r1, W_lin, b_lin, W_att, b_att,
           W_le1, b_le1, W_le2, W_le3, b_le3, W_l2, b_l2, W_r2):
    n, d = x.shape
    k = int(math.ceil(0.5 * n))
    src = edge_index[0]
    dst = edge_index[1]
    ones_e = jnp.ones(src.shape, jnp.float32)

    msum = jax.ops.segment_sum(x[src], dst, num_segments=n)
    cnt = jax.ops.segment_sum(ones_e, dst, num_segments=n)
    mean1 = msum / jnp.clip(cnt, 1.0, None)[:, None]
    h = jax.nn.relu(mean1 @ W_l1.T + b_l1 + x @ W_r1.T)

    x_q = jnp.maximum(jax.ops.segment_max(h[src], dst, num_segments=n), h)
    wq = W_att[0, :d]
    wj = W_att[0, d:]
    alpha = x_q @ (W_lin.T @ wq) + (b_lin @ wq + b_att[0])
    beta = h @ wj
    s_loop = jax.nn.leaky_relu(alpha + beta, 0.2)
    s_e = jax.nn.leaky_relu(alpha[dst] + beta[src], 0.2)
    smax = jnp.maximum(jax.ops.segment_max(s_e, dst, num_segments=n), s_loop)
    ex_e = jnp.exp(s_e - smax[dst])
    ex_loop = jnp.exp(s_loop - smax)
    den = jax.ops.segment_sum(ex_e, dst, num_segments=n) + ex_loop
    sc_e = ex_e / den[dst]
    sc_loop = ex_loop / den
    xp = (jax.ops.segment_sum(h[src] * sc_e[:, None], dst, num_segments=n)
          + h * sc_loop[:, None])

    a_n = xp @ W_le1[0] + b_le1[0]
    b_n = xp @ W_le2[0]
    agg = (jax.ops.segment_sum(a_n[src], dst, num_segments=n) + a_n
           - (cnt + 1.0) * b_n)
    fitness = jax.nn.sigmoid(agg + xp @ W_le3[0] + b_le3[0])

    _, idx = lax.top_k(fitness, k)
    sel = jnp.sort(idx)
    xsel = xp[sel] * fitness[sel][:, None]

    b_ = _BLK
    np_ = ((n + b_ - 1) // b_) * b_
    kp = ((k + b_ - 1) // b_) * b_
    loops = jnp.arange(n, dtype=src.dtype)
    row = jnp.concatenate([src, loops])
    col = jnp.concatenate([dst, loops])
    ones_t = jnp.ones(row.shape, jnp.float32)
    m_pat = jnp.zeros((np_, np_), jnp.float32).at[row, col].add(
        ones_t).astype(jnp.bfloat16)
    pos_of = jnp.full((n,), -1, jnp.int32).at[sel].set(
        jnp.arange(k, dtype=jnp.int32))
    pc = pos_of[col]
    posd = jnp.where(pc >= 0, pc, kp)
    u_pat = jnp.zeros((np_, kp + 128), jnp.float32).at[row, posd].add(
        ones_t)[:, :kp].astype(jnp.bfloat16)
    ut_pat = u_pat.T

    w_pat = _w_pattern(m_pat, u_pat)

    f = 2 * d
    xa = jnp.zeros((kp, f), jnp.float32)
    xa = xa.at[:k, :d].set(xsel)
    xa = xa.at[:k, d].set(1.0)
    wd = _mask_stage(ut_pat, w_pat, xa)
    red = _final_reduce(wd, xa, d)
    mean_sum = red[0, :d]
    xsel_sum = red[1, :d]
    out = (mean_sum @ W_l2.T + k * b_l2 + xsel_sum @ W_r2.T) / k
    return out.reshape(1, d)


# speedup vs baseline: 2.9374x; 1.2826x over previous
import functools
import math

import jax
import jax.numpy as jnp
from jax import lax
from jax.experimental import pallas as pl
from jax.experimental.pallas import tpu as pltpu

_BLK = 1024


def _w_pattern_body(m_ref, u_ref, o_ref, acc_ref, *, nr):
    r = pl.program_id(2)

    @pl.when(r == 0)
    def _init():
        acc_ref[...] = jnp.zeros_like(acc_ref)

    acc_ref[...] += jnp.dot(m_ref[...], u_ref[...],
                            preferred_element_type=jnp.float32)

    @pl.when(r == nr - 1)
    def _done():
        o_ref[...] = (acc_ref[...] > 0.0).astype(jnp.bfloat16)


def _w_pattern(m, u):
    np_, kp = m.shape[0], u.shape[1]
    b = _BLK
    ni, nj, nr = np_ // b, kp // b, np_ // b
    return pl.pallas_call(
        functools.partial(_w_pattern_body, nr=nr),
        grid=(ni, nj, nr),
        in_specs=[
            pl.BlockSpec((b, b), lambda i, j, r: (i, r)),
            pl.BlockSpec((b, b), lambda i, j, r: (r, j)),
        ],
        out_specs=pl.BlockSpec((b, b), lambda i, j, r: (i, j)),
        out_shape=jax.ShapeDtypeStruct((np_, kp), jnp.bfloat16),
        scratch_shapes=[pltpu.VMEM((b, b), jnp.float32)],
    )(m, u)


def _mask_stage_body(ut_ref, w_ref, xa_ref, o_ref, acc_ref, *, nr, b):
    j = pl.program_id(0)
    i = pl.program_id(1)
    r = pl.program_id(2)

    @pl.when(r == 0)
    def _init():
        acc_ref[...] = jnp.zeros_like(acc_ref)

    acc_ref[...] += jnp.dot(ut_ref[...], w_ref[...],
                            preferred_element_type=jnp.float32)

    @pl.when(r == nr - 1)
    def _done():
        gi = i * b + lax.broadcasted_iota(jnp.int32, (b, b), 0)
        gj = j * b + lax.broadcasted_iota(jnp.int32, (b, b), 1)
        mask = jnp.where((acc_ref[...] > 0.0) & (gi != gj), 1.0, 0.0)
        contrib = lax.dot_general(mask, xa_ref[...],
                                  (((0,), (0,)), ((), ())),
                                  preferred_element_type=jnp.float32)

        @pl.when(i == 0)
        def _set():
            o_ref[...] = contrib

        @pl.when(i != 0)
        def _add():
            o_ref[...] += contrib


def _mask_stage(ut, w, xa):
    kp, np_ = ut.shape
    f = xa.shape[1]
    b = _BLK
    nj, ni, nr = kp // b, kp // b, np_ // b
    return pl.pallas_call(
        functools.partial(_mask_stage_body, nr=nr, b=b),
        grid=(nj, ni, nr),
        in_specs=[
            pl.BlockSpec((b, b), lambda j, i, r: (i, r)),
            pl.BlockSpec((b, b), lambda j, i, r: (r, j)),
            pl.BlockSpec((b, f), lambda j, i, r: (i, 0)),
        ],
        out_specs=pl.BlockSpec((b, f), lambda j, i, r: (j, 0)),
        out_shape=jax.ShapeDtypeStruct((kp, f), jnp.float32),
        scratch_shapes=[pltpu.VMEM((b, b), jnp.float32)],
    )(ut, w, xa)


def _final_reduce_body(wd_ref, xa_ref, o_ref, *, d):
    q = pl.program_id(0)

    @pl.when(q == 0)
    def _init():
        o_ref[...] = jnp.zeros_like(o_ref)

    w = wd_ref[:, :d]
    deg = wd_ref[:, d:d + 1]
    m = w / jnp.maximum(deg, 1.0)
    o_ref[0:1, :d] += jnp.sum(m, axis=0, keepdims=True)
    o_ref[1:2, :d] += jnp.sum(xa_ref[:, :d], axis=0, keepdims=True)


def _final_reduce(wd, xa, d):
    kp, f = wd.shape
    b = _BLK
    return pl.pallas_call(
        functools.partial(_final_reduce_body, d=d),
        grid=(kp // b,),
        in_specs=[
            pl.BlockSpec((b, f), lambda q: (q, 0)),
            pl.BlockSpec((b, f), lambda q: (q, 0)),
        ],
        out_specs=pl.BlockSpec((8, f), lambda q: (0, 0)),
        out_shape=jax.ShapeDtypeStruct((8, f), jnp.float32),
    )(wd, xa)


def kernel(x, edge_index, batch, W_l1, b_l1, W_r1, W_lin, b_lin, W_att, b_att,
           W_le1, b_le1, W_le2, W_le3, b_le3, W_l2, b_l2, W_r2):
    n, d = x.shape
    k = int(math.ceil(0.5 * n))
    src = edge_index[0]
    dst = edge_index[1]
    ones_e = jnp.ones(src.shape, jnp.float32)

    msum = jax.ops.segment_sum(x[src], dst, num_segments=n)
    cnt = jax.ops.segment_sum(ones_e, dst, num_segments=n)
    mean1 = msum / jnp.clip(cnt, 1.0, None)[:, None]
    h = jax.nn.relu(mean1 @ W_l1.T + b_l1 + x @ W_r1.T)

    hs = h[src]
    x_q = jnp.maximum(jax.ops.segment_max(hs, dst, num_segments=n), h)
    wq = W_att[0, :d]
    wj = W_att[0, d:]
    alpha = x_q @ (W_lin.T @ wq) + (b_lin @ wq + b_att[0])
    beta = h @ wj
    s_loop = jax.nn.leaky_relu(alpha + beta, 0.2)
    s_e = jax.nn.leaky_relu(alpha[dst] + beta[src], 0.2)
    ex_e = jnp.exp(s_e)
    ex_loop = jnp.exp(s_loop)
    den = jax.ops.segment_sum(ex_e, dst, num_segments=n) + ex_loop
    xp = (jax.ops.segment_sum(hs * ex_e[:, None], dst, num_segments=n)
          + h * ex_loop[:, None]) / den[:, None]

    a_n = xp @ W_le1[0] + b_le1[0]
    b_n = xp @ W_le2[0]
    agg = (jax.ops.segment_sum(a_n[src], dst, num_segments=n) + a_n
           - (cnt + 1.0) * b_n)
    fitness = jax.nn.sigmoid(agg + xp @ W_le3[0] + b_le3[0])

    _, idx = lax.top_k(fitness, k)
    sel = jnp.sort(idx)
    xsel = xp[sel] * fitness[sel][:, None]

    b_ = _BLK
    np_ = ((n + b_ - 1) // b_) * b_
    kp = ((k + b_ - 1) // b_) * b_
    loops = jnp.arange(n, dtype=src.dtype)
    row = jnp.concatenate([src, loops])
    col = jnp.concatenate([dst, loops])
    ones_t = jnp.ones(row.shape, jnp.float32)
    m_pat = jnp.zeros((np_, np_), jnp.float32).at[row, col].add(
        ones_t).astype(jnp.bfloat16)
    pos_of = jnp.full((n,), -1, jnp.int32).at[sel].set(
        jnp.arange(k, dtype=jnp.int32))
    pc = pos_of[col]
    posd = jnp.where(pc >= 0, pc, kp)
    u_pat = jnp.zeros((np_, kp + 128), jnp.float32).at[row, posd].add(
        ones_t)[:, :kp].astype(jnp.bfloat16)
    ut_pat = u_pat.T

    w_pat = _w_pattern(m_pat, u_pat)

    f = 2 * d
    xa = jnp.zeros((kp, f), jnp.float32)
    xa = xa.at[:k, :d].set(xsel)
    xa = xa.at[:k, d].set(1.0)
    wd = _mask_stage(ut_pat, w_pat, xa)
    red = _final_reduce(wd, xa, d)
    mean_sum = red[0, :d]
    xsel_sum = red[1, :d]
    out = (mean_sum @ W_l2.T + k * b_l2 + xsel_sum @ W_r2.T) / k
    return out.reshape(1, d)


# speedup vs baseline: 3.6436x; 1.2404x over previous
import functools
import math

import jax
import jax.numpy as jnp
from jax import lax
from jax.experimental import pallas as pl
from jax.experimental.pallas import tpu as pltpu
from jax.experimental.pallas import tpu_sc as plsc

_BLK = 1024
_EB = 128


def _sc_gather_segsum(table, idx, seg, w):
    n, d = table.shape
    e_tot = idx.shape[0]
    info = plsc.get_sparse_core_info()
    nc, ns = info.num_cores, info.num_subcores
    nw = nc * ns
    full = e_tot // (nw * _EB)
    rem = e_tot - nw * full * _EB
    assert rem % _EB == 0 and rem // _EB <= nw
    rem_blocks = rem // _EB
    n_pad = ((n + 8 * ns - 1) // (8 * ns)) * (8 * ns)
    rows_per_tile = n_pad // ns
    have_w = w is not None
    mesh = plsc.VectorSubcoreMesh(core_axis_name="c", subcore_axis_name="s")

    def body(*refs):
        if have_w:
            (table_h, idx_h, seg_h, w_h, zero_h, out_h,
             acc_sh, idx_v, seg_v, w_v, rows_v, sem) = refs
        else:
            (table_h, idx_h, seg_h, zero_h, out_h,
             acc_sh, idx_v, seg_v, rows_v, sem) = refs
            w_v = None
        cid = lax.axis_index("c")
        sid = lax.axis_index("s")
        wid = sid * nc + cid
        pltpu.sync_copy(zero_h, acc_sh.at[pl.ds(sid * rows_per_tile,
                                                rows_per_tile)])
        plsc.subcore_barrier()

        def do_block(base):
            pltpu.sync_copy(idx_h.at[pl.ds(base, _EB)], idx_v)
            pltpu.sync_copy(seg_h.at[pl.ds(base, _EB)], seg_v)
            pltpu.async_copy(table_h.at[idx_v], rows_v, sem).wait()
            if have_w:
                pltpu.sync_copy(w_h.at[pl.ds(base, _EB)], w_v)

                def scale(eb, _):
                    wv = plsc.load_gather(
                        w_v, [jnp.full((16,), eb, jnp.int32)])
                    for kk in range(d // 16):
                        sl = pl.ds(kk * 16, 16)
                        rows_v[eb, sl] = rows_v[eb, sl] * wv
                    return ()

                lax.fori_loop(0, _EB, scale, (), unroll=False)
            pltpu.sync_copy(rows_v, acc_sh.at[seg_v], add=True)

        tile_base = wid * full * _EB

        def blk(b, _):
            do_block(tile_base + b * _EB)
            return ()

        lax.fori_loop(0, full, blk, (), unroll=False)

        @pl.when(wid < rem_blocks)
        def _tail():
            do_block(nw * full * _EB + wid * _EB)

        plsc.subcore_barrier()
        pltpu.sync_copy(
            acc_sh.at[pl.ds(sid * rows_per_tile, rows_per_tile)],
            out_h.at[cid, pl.ds(sid * rows_per_tile, rows_per_tile)])

    scratch = [
        pltpu.VMEM_SHARED((n_pad, d), jnp.float32),
        pltpu.VMEM((_EB,), jnp.int32),
        pltpu.VMEM((_EB,), jnp.int32),
    ]
    if have_w:
        scratch.append(pltpu.VMEM((_EB,), jnp.float32))
    scratch += [
        pltpu.VMEM((_EB, d), jnp.float32),
        pltpu.SemaphoreType.DMA,
    ]
    zero = jnp.zeros((rows_per_tile, d), jnp.float32)
    args = (table, idx, seg, w, zero) if have_w else (table, idx, seg, zero)
    out = pl.kernel(
        body,
        out_type=jax.ShapeDtypeStruct((nc, n_pad, d), jnp.float32),
        mesh=mesh,
        scratch_types=scratch,
        compiler_params=pltpu.CompilerParams(needs_layout_passes=False),
    )(*args)
    return out[0, :n] + out[1, :n]


def _w_pattern_body(m_ref, u_ref, o_ref, acc_ref, *, nr):
    r = pl.program_id(2)

    @pl.when(r == 0)
    def _init():
        acc_ref[...] = jnp.zeros_like(acc_ref)

    acc_ref[...] += jnp.dot(m_ref[...], u_ref[...],
                            preferred_element_type=jnp.float32)

    @pl.when(r == nr - 1)
    def _done():
        o_ref[...] = (acc_ref[...] > 0.0).astype(jnp.bfloat16)


def _w_pattern(m, u):
    np_, kp = m.shape[0], u.shape[1]
    b = _BLK
    ni, nj, nr = np_ // b, kp // b, np_ // b
    return pl.pallas_call(
        functools.partial(_w_pattern_body, nr=nr),
        grid=(ni, nj, nr),
        in_specs=[
            pl.BlockSpec((b, b), lambda i, j, r: (i, r)),
            pl.BlockSpec((b, b), lambda i, j, r: (r, j)),
        ],
        out_specs=pl.BlockSpec((b, b), lambda i, j, r: (i, j)),
        out_shape=jax.ShapeDtypeStruct((np_, kp), jnp.bfloat16),
        scratch_shapes=[pltpu.VMEM((b, b), jnp.float32)],
    )(m, u)


def _mask_stage_body(ut_ref, w_ref, xa_ref, o_ref, acc_ref, *, nr, b):
    j = pl.program_id(0)
    i = pl.program_id(1)
    r = pl.program_id(2)

    @pl.when(r == 0)
    def _init():
        acc_ref[...] = jnp.zeros_like(acc_ref)

    acc_ref[...] += jnp.dot(ut_ref[...], w_ref[...],
                            preferred_element_type=jnp.float32)

    @pl.when(r == nr - 1)
    def _done():
        gi = i * b + lax.broadcasted_iota(jnp.int32, (b, b), 0)
        gj = j * b + lax.broadcasted_iota(jnp.int32, (b, b), 1)
        mask = jnp.where((acc_ref[...] > 0.0) & (gi != gj), 1.0, 0.0)
        contrib = lax.dot_general(mask, xa_ref[...],
                                  (((0,), (0,)), ((), ())),
                                  preferred_element_type=jnp.float32)

        @pl.when(i == 0)
        def _set():
            o_ref[...] = contrib

        @pl.when(i != 0)
        def _add():
            o_ref[...] += contrib


def _mask_stage(ut, w, xa):
    kp, np_ = ut.shape
    f = xa.shape[1]
    b = _BLK
    nj, ni, nr = kp // b, kp // b, np_ // b
    return pl.pallas_call(
        functools.partial(_mask_stage_body, nr=nr, b=b),
        grid=(nj, ni, nr),
        in_specs=[
            pl.BlockSpec((b, b), lambda j, i, r: (i, r)),
            pl.BlockSpec((b, b), lambda j, i, r: (r, j)),
            pl.BlockSpec((b, f), lambda j, i, r: (i, 0)),
        ],
        out_specs=pl.BlockSpec((b, f), lambda j, i, r: (j, 0)),
        out_shape=jax.ShapeDtypeStruct((kp, f), jnp.float32),
        scratch_shapes=[pltpu.VMEM((b, b), jnp.float32)],
    )(ut, w, xa)


def _final_reduce_body(wd_ref, xa_ref, o_ref, *, d):
    q = pl.program_id(0)

    @pl.when(q == 0)
    def _init():
        o_ref[...] = jnp.zeros_like(o_ref)

    w = wd_ref[:, :d]
    deg = wd_ref[:, d:d + 1]
    m = w / jnp.maximum(deg, 1.0)
    o_ref[0:1, :d] += jnp.sum(m, axis=0, keepdims=True)
    o_ref[1:2, :d] += jnp.sum(xa_ref[:, :d], axis=0, keepdims=True)


def _final_reduce(wd, xa, d):
    kp, f = wd.shape
    b = _BLK
    return pl.pallas_call(
        functools.partial(_final_reduce_body, d=d),
        grid=(kp // b,),
        in_specs=[
            pl.BlockSpec((b, f), lambda q: (q, 0)),
            pl.BlockSpec((b, f), lambda q: (q, 0)),
        ],
        out_specs=pl.BlockSpec((8, f), lambda q: (0, 0)),
        out_shape=jax.ShapeDtypeStruct((8, f), jnp.float32),
    )(wd, xa)


def kernel(x, edge_index, batch, W_l1, b_l1, W_r1, W_lin, b_lin, W_att, b_att,
           W_le1, b_le1, W_le2, W_le3, b_le3, W_l2, b_l2, W_r2):
    n, d = x.shape
    k = int(math.ceil(0.5 * n))
    src = edge_index[0]
    dst = edge_index[1]
    ones_e = jnp.ones(src.shape, jnp.float32)

    msum = _sc_gather_segsum(x, src, dst, None)
    cnt = jax.ops.segment_sum(ones_e, dst, num_segments=n)
    mean1 = msum / jnp.clip(cnt, 1.0, None)[:, None]
    h = jax.nn.relu(mean1 @ W_l1.T + b_l1 + x @ W_r1.T)

    hs = h[src]
    x_q = jnp.maximum(jax.ops.segment_max(hs, dst, num_segments=n), h)
    wq = W_att[0, :d]
    wj = W_att[0, d:]
    alpha = x_q @ (W_lin.T @ wq) + (b_lin @ wq + b_att[0])
    beta = h @ wj
    s_loop = jax.nn.leaky_relu(alpha + beta, 0.2)
    s_e = jax.nn.leaky_relu(alpha[dst] + beta[src], 0.2)
    ex_e = jnp.exp(s_e)
    ex_loop = jnp.exp(s_loop)
    den = jax.ops.segment_sum(ex_e, dst, num_segments=n) + ex_loop
    xp = (_sc_gather_segsum(h, src, dst, ex_e)
          + h * ex_loop[:, None]) / den[:, None]

    a_n = xp @ W_le1[0] + b_le1[0]
    b_n = xp @ W_le2[0]
    agg = (jax.ops.segment_sum(a_n[src], dst, num_segments=n) + a_n
           - (cnt + 1.0) * b_n)
    fitness = jax.nn.sigmoid(agg + xp @ W_le3[0] + b_le3[0])

    _, idx = lax.top_k(fitness, k)
    sel = jnp.sort(idx)
    xsel = xp[sel] * fitness[sel][:, None]

    b_ = _BLK
    np_ = ((n + b_ - 1) // b_) * b_
    kp = ((k + b_ - 1) // b_) * b_
    loops = jnp.arange(n, dtype=src.dtype)
    row = jnp.concatenate([src, loops])
    col = jnp.concatenate([dst, loops])
    ones_t = jnp.ones(row.shape, jnp.float32)
    m_pat = jnp.zeros((np_, np_), jnp.float32).at[row, col].add(
        ones_t).astype(jnp.bfloat16)
    pos_of = jnp.full((n,), -1, jnp.int32).at[sel].set(
        jnp.arange(k, dtype=jnp.int32))
    pc = pos_of[col]
    posd = jnp.where(pc >= 0, pc, kp)
    u_pat = jnp.zeros((np_, kp + 128), jnp.float32).at[row, posd].add(
        ones_t)[:, :kp].astype(jnp.bfloat16)
    ut_pat = u_pat.T

    w_pat = _w_pattern(m_pat, u_pat)

    f = 2 * d
    xa = jnp.zeros((kp, f), jnp.float32)
    xa = xa.at[:k, :d].set(xsel)
    xa = xa.at[:k, d].set(1.0)
    wd = _mask_stage(ut_pat, w_pat, xa)
    red = _final_reduce(wd, xa, d)
    mean_sum = red[0, :d]
    xsel_sum = red[1, :d]
    out = (mean_sum @ W_l2.T + k * b_l2 + xsel_sum @ W_r2.T) / k
    return out.reshape(1, d)


# speedup vs baseline: 6.2347x; 1.7112x over previous
import functools
import math

import jax
import jax.numpy as jnp
from jax import lax
from jax.experimental import pallas as pl
from jax.experimental.pallas import tpu as pltpu
from jax.experimental.pallas import tpu_sc as plsc

_BLK = 1024
_EB = 128


def _leaky_exp(z):
    return jnp.exp(jnp.maximum(z, 0.2 * z))


def _sc_gather_segsum(table, idx, seg, alpha, beta):
    n, d = table.shape
    e_tot = idx.shape[0]
    info = plsc.get_sparse_core_info()
    nc, ns = info.num_cores, info.num_subcores
    nw = nc * ns
    full = e_tot // (nw * _EB)
    rem = e_tot - nw * full * _EB
    assert rem % _EB == 0 and rem // _EB <= nw
    rem_blocks = rem // _EB
    n_pad = ((n + 8 * ns - 1) // (8 * ns)) * (8 * ns)
    rows_per_tile = n_pad // ns
    have_w = alpha is not None
    mesh = plsc.VectorSubcoreMesh(core_axis_name="c", subcore_axis_name="s")

    def body(*refs):
        if have_w:
            (table_h, idx_h, seg_h, al_h, be_h, zero_h, out_h, outs_h,
             acc_sh, idx_v, seg_v, w_v, rows_v, al_v, be_v, sacc_v,
             sem) = refs
        else:
            (table_h, idx_h, seg_h, zero_h, out_h, outs_h,
             acc_sh, idx_v, seg_v, rows_v, sacc_v, sem) = refs
        cid = lax.axis_index("c")
        sid = lax.axis_index("s")
        wid = sid * nc + cid
        pltpu.sync_copy(zero_h, acc_sh.at[pl.ds(sid * rows_per_tile,
                                                rows_per_tile)])
        if have_w:
            pltpu.sync_copy(al_h, al_v.at[pl.ds(0, n)])
            pltpu.sync_copy(be_h, be_v.at[pl.ds(0, n)])

        def zr(i, _):
            sacc_v[pl.ds(i * 16, 16)] = jnp.zeros((16,), jnp.float32)
            return ()

        lax.fori_loop(0, n_pad // 16, zr, (), unroll=False)
        plsc.subcore_barrier()

        def do_block(base):
            pltpu.sync_copy(idx_h.at[pl.ds(base, _EB)], idx_v)
            pltpu.sync_copy(seg_h.at[pl.ds(base, _EB)], seg_v)
            pltpu.async_copy(table_h.at[idx_v], rows_v, sem).wait()
            ones16 = jnp.ones((16,), jnp.float32)
            for j in range(_EB // 16):
                sl = pl.ds(j * 16, 16)
                s16 = seg_v[sl]
                if have_w:
                    av = plsc.load_gather(al_v, [s16])
                    bv = plsc.load_gather(be_v, [idx_v[sl]])
                    ex = _leaky_exp(av + bv)
                    w_v[sl] = ex
                else:
                    ex = ones16
                plsc.addupdate_scatter(sacc_v, [s16], ex)
            if have_w:
                def scale(eb, _):
                    wv = plsc.load_gather(
                        w_v, [jnp.full((16,), eb, jnp.int32)])
                    for kk in range(d // 16):
                        sl2 = pl.ds(kk * 16, 16)
                        rows_v[eb, sl2] = rows_v[eb, sl2] * wv
                    return ()

                lax.fori_loop(0, _EB, scale, (), unroll=False)
            pltpu.sync_copy(rows_v, acc_sh.at[seg_v], add=True)

        tile_base = wid * full * _EB

        def blk(b, _):
            do_block(tile_base + b * _EB)
            return ()

        lax.fori_loop(0, full, blk, (), unroll=False)

        @pl.when(wid < rem_blocks)
        def _tail():
            do_block(nw * full * _EB + wid * _EB)

        plsc.subcore_barrier()
        pltpu.sync_copy(
            acc_sh.at[pl.ds(sid * rows_per_tile, rows_per_tile)],
            out_h.at[cid, pl.ds(sid * rows_per_tile, rows_per_tile)])
        pltpu.sync_copy(sacc_v, outs_h.at[cid, sid])

    scratch = [
        pltpu.VMEM_SHARED((n_pad, d), jnp.float32),
        pltpu.VMEM((_EB,), jnp.int32),
        pltpu.VMEM((_EB,), jnp.int32),
    ]
    if have_w:
        scratch.append(pltpu.VMEM((_EB,), jnp.float32))
    scratch.append(pltpu.VMEM((_EB, d), jnp.float32))
    if have_w:
        scratch += [pltpu.VMEM((n_pad,), jnp.float32),
                    pltpu.VMEM((n_pad,), jnp.float32)]
    scratch += [
        pltpu.VMEM((n_pad,), jnp.float32),
        pltpu.SemaphoreType.DMA,
    ]
    zero = jnp.zeros((rows_per_tile, d), jnp.float32)
    args = ((table, idx, seg, alpha, beta, zero) if have_w
            else (table, idx, seg, zero))
    vec, scal = pl.kernel(
        body,
        out_type=(jax.ShapeDtypeStruct((nc, n_pad, d), jnp.float32),
                  jax.ShapeDtypeStruct((nc, ns, n_pad), jnp.float32)),
        mesh=mesh,
        scratch_types=scratch,
        compiler_params=pltpu.CompilerParams(needs_layout_passes=False),
    )(*args)
    return vec[0, :n] + vec[1, :n], jnp.sum(scal, axis=(0, 1))[:n]


def _sc_scalar_segsum(vals, idx, seg, n):
    e_tot = idx.shape[0]
    info = plsc.get_sparse_core_info()
    nc, ns = info.num_cores, info.num_subcores
    nw = nc * ns
    full = e_tot // (nw * _EB)
    rem = e_tot - nw * full * _EB
    assert rem % _EB == 0 and rem // _EB <= nw
    rem_blocks = rem // _EB
    n_pad = ((n + 8 * ns - 1) // (8 * ns)) * (8 * ns)
    mesh = plsc.VectorSubcoreMesh(core_axis_name="c", subcore_axis_name="s")

    def body(vals_h, idx_h, seg_h, outs_h, vals_v, idx_v, seg_v, sacc_v):
        cid = lax.axis_index("c")
        sid = lax.axis_index("s")
        wid = sid * nc + cid
        pltpu.sync_copy(vals_h, vals_v.at[pl.ds(0, n)])

        def zr(i, _):
            sacc_v[pl.ds(i * 16, 16)] = jnp.zeros((16,), jnp.float32)
            return ()

        lax.fori_loop(0, n_pad // 16, zr, (), unroll=False)

        def do_block(base):
            pltpu.sync_copy(idx_h.at[pl.ds(base, _EB)], idx_v)
            pltpu.sync_copy(seg_h.at[pl.ds(base, _EB)], seg_v)
            for j in range(_EB // 16):
                sl = pl.ds(j * 16, 16)
                v = plsc.load_gather(vals_v, [idx_v[sl]])
                plsc.addupdate_scatter(sacc_v, [seg_v[sl]], v)

        def blk(b, _):
            do_block(wid * full * _EB + b * _EB)
            return ()

        lax.fori_loop(0, full, blk, (), unroll=False)

        @pl.when(wid < rem_blocks)
        def _tail():
            do_block(nw * full * _EB + wid * _EB)

        pltpu.sync_copy(sacc_v, outs_h.at[cid, sid])

    out = pl.kernel(
        body,
        out_type=jax.ShapeDtypeStruct((nc, ns, n_pad), jnp.float32),
        mesh=mesh,
        scratch_types=[
            pltpu.VMEM((n_pad,), jnp.float32),
            pltpu.VMEM((_EB,), jnp.int32),
            pltpu.VMEM((_EB,), jnp.int32),
            pltpu.VMEM((n_pad,), jnp.float32),
        ],
        compiler_params=pltpu.CompilerParams(needs_layout_passes=False),
    )(vals, idx, seg)
    return jnp.sum(out, axis=(0, 1))[:n]


def _w_pattern_body(m_ref, u_ref, o_ref, acc_ref, *, nr):
    r = pl.program_id(2)

    @pl.when(r == 0)
    def _init():
        acc_ref[...] = jnp.zeros_like(acc_ref)

    acc_ref[...] += jnp.dot(m_ref[...], u_ref[...],
                            preferred_element_type=jnp.float32)

    @pl.when(r == nr - 1)
    def _done():
        o_ref[...] = (acc_ref[...] > 0.0).astype(jnp.bfloat16)


def _w_pattern(m, u):
    np_, kp = m.shape[0], u.shape[1]
    b = _BLK
    ni, nj, nr = np_ // b, kp // b, np_ // b
    return pl.pallas_call(
        functools.partial(_w_pattern_body, nr=nr),
        grid=(ni, nj, nr),
        in_specs=[
            pl.BlockSpec((b, b), lambda i, j, r: (i, r)),
            pl.BlockSpec((b, b), lambda i, j, r: (r, j)),
        ],
        out_specs=pl.BlockSpec((b, b), lambda i, j, r: (i, j)),
        out_shape=jax.ShapeDtypeStruct((np_, kp), jnp.bfloat16),
        scratch_shapes=[pltpu.VMEM((b, b), jnp.float32)],
    )(m, u)


def _mask_stage_body(ut_ref, w_ref, xa_ref, o_ref, acc_ref, *, nr, b):
    j = pl.program_id(0)
    i = pl.program_id(1)
    r = pl.program_id(2)

    @pl.when(r == 0)
    def _init():
        acc_ref[...] = jnp.zeros_like(acc_ref)

    acc_ref[...] += jnp.dot(ut_ref[...], w_ref[...],
                            preferred_element_type=jnp.float32)

    @pl.when(r == nr - 1)
    def _done():
        gi = i * b + lax.broadcasted_iota(jnp.int32, (b, b), 0)
        gj = j * b + lax.broadcasted_iota(jnp.int32, (b, b), 1)
        mask = jnp.where((acc_ref[...] > 0.0) & (gi != gj), 1.0, 0.0)
        contrib = lax.dot_general(mask, xa_ref[...],
                                  (((0,), (0,)), ((), ())),
                                  preferred_element_type=jnp.float32)

        @pl.when(i == 0)
        def _set():
            o_ref[...] = contrib

        @pl.when(i != 0)
        def _add():
            o_ref[...] += contrib


def _mask_stage(ut, w, xa):
    kp, np_ = ut.shape
    f = xa.shape[1]
    b = _BLK
    nj, ni, nr = kp // b, kp // b, np_ // b
    return pl.pallas_call(
        functools.partial(_mask_stage_body, nr=nr, b=b),
        grid=(nj, ni, nr),
        in_specs=[
            pl.BlockSpec((b, b), lambda j, i, r: (i, r)),
            pl.BlockSpec((b, b), lambda j, i, r: (r, j)),
            pl.BlockSpec((b, f), lambda j, i, r: (i, 0)),
        ],
        out_specs=pl.BlockSpec((b, f), lambda j, i, r: (j, 0)),
        out_shape=jax.ShapeDtypeStruct((kp, f), jnp.float32),
        scratch_shapes=[pltpu.VMEM((b, b), jnp.float32)],
    )(ut, w, xa)


def _final_reduce_body(wd_ref, xa_ref, o_ref, *, d):
    q = pl.program_id(0)

    @pl.when(q == 0)
    def _init():
        o_ref[...] = jnp.zeros_like(o_ref)

    w = wd_ref[:, :d]
    deg = wd_ref[:, d:d + 1]
    m = w / jnp.maximum(deg, 1.0)
    o_ref[0:1, :d] += jnp.sum(m, axis=0, keepdims=True)
    o_ref[1:2, :d] += jnp.sum(xa_ref[:, :d], axis=0, keepdims=True)


def _final_reduce(wd, xa, d):
    kp, f = wd.shape
    b = _BLK
    return pl.pallas_call(
        functools.partial(_final_reduce_body, d=d),
        grid=(kp // b,),
        in_specs=[
            pl.BlockSpec((b, f), lambda q: (q, 0)),
            pl.BlockSpec((b, f), lambda q: (q, 0)),
        ],
        out_specs=pl.BlockSpec((8, f), lambda q: (0, 0)),
        out_shape=jax.ShapeDtypeStruct((8, f), jnp.float32),
    )(wd, xa)


def kernel(x, edge_index, batch, W_l1, b_l1, W_r1, W_lin, b_lin, W_att, b_att,
           W_le1, b_le1, W_le2, W_le3, b_le3, W_l2, b_l2, W_r2):
    n, d = x.shape
    k = int(math.ceil(0.5 * n))
    src = edge_index[0]
    dst = edge_index[1]
    ones_e = jnp.ones(src.shape, jnp.float32)

    msum, cnt = _sc_gather_segsum(x, src, dst, None, None)
    mean1 = msum / jnp.clip(cnt, 1.0, None)[:, None]
    h = jax.nn.relu(mean1 @ W_l1.T + b_l1 + x @ W_r1.T)

    hs = h[src]
    x_q = jnp.maximum(jax.ops.segment_max(hs, dst, num_segments=n), h)
    wq = W_att[0, :d]
    wj = W_att[0, d:]
    alpha = x_q @ (W_lin.T @ wq) + (b_lin @ wq + b_att[0])
    beta = h @ wj
    ex_loop = _leaky_exp(alpha + beta)
    xp_num, den_e = _sc_gather_segsum(h, src, dst, alpha, beta)
    den = den_e + ex_loop
    xp = (xp_num + h * ex_loop[:, None]) / den[:, None]

    a_n = xp @ W_le1[0] + b_le1[0]
    b_n = xp @ W_le2[0]
    agg = (_sc_scalar_segsum(a_n, src, dst, n) + a_n
           - (cnt + 1.0) * b_n)
    fitness = jax.nn.sigmoid(agg + xp @ W_le3[0] + b_le3[0])

    _, idx = lax.top_k(fitness, k)
    sel = jnp.sort(idx)
    xsel = xp[sel] * fitness[sel][:, None]

    b_ = _BLK
    np_ = ((n + b_ - 1) // b_) * b_
    kp = ((k + b_ - 1) // b_) * b_
    loops = jnp.arange(n, dtype=src.dtype)
    row = jnp.concatenate([src, loops])
    col = jnp.concatenate([dst, loops])
    ones_t = jnp.ones(row.shape, jnp.float32)
    m_pat = jnp.zeros((np_, np_), jnp.float32).at[row, col].add(
        ones_t).astype(jnp.bfloat16)
    pos_of = jnp.full((n,), -1, jnp.int32).at[sel].set(
        jnp.arange(k, dtype=jnp.int32))
    pc = pos_of[col]
    posd = jnp.where(pc >= 0, pc, kp)
    u_pat = jnp.zeros((np_, kp + 128), jnp.float32).at[row, posd].add(
        ones_t)[:, :kp].astype(jnp.bfloat16)
    ut_pat = u_pat.T

    w_pat = _w_pattern(m_pat, u_pat)

    f = 2 * d
    xa = jnp.zeros((kp, f), jnp.float32)
    xa = xa.at[:k, :d].set(xsel)
    xa = xa.at[:k, d].set(1.0)
    wd = _mask_stage(ut_pat, w_pat, xa)
    red = _final_reduce(wd, xa, d)
    mean_sum = red[0, :d]
    xsel_sum = red[1, :d]
    out = (mean_sum @ W_l2.T + k * b_l2 + xsel_sum @ W_r2.T) / k
    return out.reshape(1, d)


# speedup vs baseline: 6.3064x; 1.0115x over previous
import functools
import math

import jax
import jax.numpy as jnp
from jax import lax
from jax.experimental import pallas as pl
from jax.experimental.pallas import tpu as pltpu
from jax.experimental.pallas import tpu_sc as plsc

_BLK = 1024
_EB = 128


def _leaky_exp(z):
    return jnp.exp(jnp.maximum(z, 0.2 * z))


def _sc_gather_segsum(table, idx, seg, alpha, beta):
    n, d = table.shape
    e_tot = idx.shape[0]
    info = plsc.get_sparse_core_info()
    nc, ns = info.num_cores, info.num_subcores
    nw = nc * ns
    full = e_tot // (nw * _EB)
    rem = e_tot - nw * full * _EB
    assert rem % _EB == 0 and rem // _EB <= nw
    rem_blocks = rem // _EB
    n_pad = ((n + 8 * ns - 1) // (8 * ns)) * (8 * ns)
    rows_per_tile = n_pad // ns
    have_w = alpha is not None
    mesh = plsc.VectorSubcoreMesh(core_axis_name="c", subcore_axis_name="s")

    def body(*refs):
        if have_w:
            (table_h, idx_h, seg_h, al_h, be_h, zero_h, out_h, outs_h,
             acc_sh, idx_v, seg_v, w_v, rows_v, al_v, be_v, sacc_v,
             sem) = refs
        else:
            (table_h, idx_h, seg_h, zero_h, out_h, outs_h,
             acc_sh, idx_v, seg_v, rows_v, sacc_v, sem) = refs
        cid = lax.axis_index("c")
        sid = lax.axis_index("s")
        wid = sid * nc + cid
        pltpu.sync_copy(zero_h, acc_sh.at[pl.ds(sid * rows_per_tile,
                                                rows_per_tile)])
        if have_w:
            pltpu.sync_copy(al_h, al_v.at[pl.ds(0, n)])
            pltpu.sync_copy(be_h, be_v.at[pl.ds(0, n)])

        def zr(i, _):
            sacc_v[pl.ds(i * 16, 16)] = jnp.zeros((16,), jnp.float32)
            return ()

        lax.fori_loop(0, n_pad // 16, zr, (), unroll=False)
        plsc.subcore_barrier()

        def do_block(base):
            pltpu.sync_copy(idx_h.at[pl.ds(base, _EB)], idx_v)
            pltpu.sync_copy(seg_h.at[pl.ds(base, _EB)], seg_v)
            pltpu.async_copy(table_h.at[idx_v], rows_v, sem).wait()
            ones16 = jnp.ones((16,), jnp.float32)
            for j in range(_EB // 16):
                sl = pl.ds(j * 16, 16)
                s16 = seg_v[sl]
                if have_w:
                    av = plsc.load_gather(al_v, [s16])
                    bv = plsc.load_gather(be_v, [idx_v[sl]])
                    ex = _leaky_exp(av + bv)
                    w_v[sl] = ex
                else:
                    ex = ones16
                plsc.addupdate_scatter(sacc_v, [s16], ex)
            if have_w:
                def scale(eb, _):
                    wv = plsc.load_gather(
                        w_v, [jnp.full((16,), eb, jnp.int32)])
                    for kk in range(d // 16):
                        sl2 = pl.ds(kk * 16, 16)
                        rows_v[eb, sl2] = rows_v[eb, sl2] * wv
                    return ()

                lax.fori_loop(0, _EB, scale, (), unroll=False)
            pltpu.sync_copy(rows_v, acc_sh.at[seg_v], add=True)

        tile_base = wid * full * _EB

        def blk(b, _):
            do_block(tile_base + b * _EB)
            return ()

        lax.fori_loop(0, full, blk, (), unroll=False)

        @pl.when(wid < rem_blocks)
        def _tail():
            do_block(nw * full * _EB + wid * _EB)

        plsc.subcore_barrier()
        pltpu.sync_copy(
            acc_sh.at[pl.ds(sid * rows_per_tile, rows_per_tile)],
            out_h.at[cid, pl.ds(sid * rows_per_tile, rows_per_tile)])
        pltpu.sync_copy(sacc_v, outs_h.at[cid, sid])

    scratch = [
        pltpu.VMEM_SHARED((n_pad, d), jnp.float32),
        pltpu.VMEM((_EB,), jnp.int32),
        pltpu.VMEM((_EB,), jnp.int32),
    ]
    if have_w:
        scratch.append(pltpu.VMEM((_EB,), jnp.float32))
    scratch.append(pltpu.VMEM((_EB, d), jnp.float32))
    if have_w:
        scratch += [pltpu.VMEM((n_pad,), jnp.float32),
                    pltpu.VMEM((n_pad,), jnp.float32)]
    scratch += [
        pltpu.VMEM((n_pad,), jnp.float32),
        pltpu.SemaphoreType.DMA,
    ]
    zero = jnp.zeros((rows_per_tile, d), jnp.float32)
    args = ((table, idx, seg, alpha, beta, zero) if have_w
            else (table, idx, seg, zero))
    vec, scal = pl.kernel(
        body,
        out_type=(jax.ShapeDtypeStruct((nc, n_pad, d), jnp.float32),
                  jax.ShapeDtypeStruct((nc, ns, n_pad), jnp.float32)),
        mesh=mesh,
        scratch_types=scratch,
        compiler_params=pltpu.CompilerParams(needs_layout_passes=False),
    )(*args)
    return vec[0, :n] + vec[1, :n], jnp.sum(scal, axis=(0, 1))[:n]


def _sc_scalar_segsum(vals, idx, seg, n):
    e_tot = idx.shape[0]
    info = plsc.get_sparse_core_info()
    nc, ns = info.num_cores, info.num_subcores
    nw = nc * ns
    full = e_tot // (nw * _EB)
    rem = e_tot - nw * full * _EB
    assert rem % _EB == 0 and rem // _EB <= nw
    rem_blocks = rem // _EB
    n_pad = ((n + 8 * ns - 1) // (8 * ns)) * (8 * ns)
    mesh = plsc.VectorSubcoreMesh(core_axis_name="c", subcore_axis_name="s")

    def body(vals_h, idx_h, seg_h, outs_h, vals_v, idx_v, seg_v, sacc_v):
        cid = lax.axis_index("c")
        sid = lax.axis_index("s")
        wid = sid * nc + cid
        pltpu.sync_copy(vals_h, vals_v.at[pl.ds(0, n)])

        def zr(i, _):
            sacc_v[pl.ds(i * 16, 16)] = jnp.zeros((16,), jnp.float32)
            return ()

        lax.fori_loop(0, n_pad // 16, zr, (), unroll=False)

        def do_block(base):
            pltpu.sync_copy(idx_h.at[pl.ds(base, _EB)], idx_v)
            pltpu.sync_copy(seg_h.at[pl.ds(base, _EB)], seg_v)
            for j in range(_EB // 16):
                sl = pl.ds(j * 16, 16)
                v = plsc.load_gather(vals_v, [idx_v[sl]])
                plsc.addupdate_scatter(sacc_v, [seg_v[sl]], v)

        def blk(b, _):
            do_block(wid * full * _EB + b * _EB)
            return ()

        lax.fori_loop(0, full, blk, (), unroll=False)

        @pl.when(wid < rem_blocks)
        def _tail():
            do_block(nw * full * _EB + wid * _EB)

        pltpu.sync_copy(sacc_v, outs_h.at[cid, sid])

    out = pl.kernel(
        body,
        out_type=jax.ShapeDtypeStruct((nc, ns, n_pad), jnp.float32),
        mesh=mesh,
        scratch_types=[
            pltpu.VMEM((n_pad,), jnp.float32),
            pltpu.VMEM((_EB,), jnp.int32),
            pltpu.VMEM((_EB,), jnp.int32),
            pltpu.VMEM((n_pad,), jnp.float32),
        ],
        compiler_params=pltpu.CompilerParams(needs_layout_passes=False),
    )(vals, idx, seg)
    return jnp.sum(out, axis=(0, 1))[:n]


def _w_pattern_body(m_ref, u_ref, o_ref, acc_ref, *, nr):
    r = pl.program_id(2)

    @pl.when(r == 0)
    def _init():
        acc_ref[...] = jnp.zeros_like(acc_ref)

    acc_ref[...] += jnp.dot(m_ref[...], u_ref[...],
                            preferred_element_type=jnp.float32)

    @pl.when(r == nr - 1)
    def _done():
        o_ref[...] = (acc_ref[...] > 0.0).astype(jnp.bfloat16)


def _w_pattern(m, u):
    np_, kp = m.shape[0], u.shape[1]
    b = _BLK
    ni, nj, nr = np_ // b, kp // b, np_ // b
    return pl.pallas_call(
        functools.partial(_w_pattern_body, nr=nr),
        grid=(ni, nj, nr),
        in_specs=[
            pl.BlockSpec((b, b), lambda i, j, r: (i, r)),
            pl.BlockSpec((b, b), lambda i, j, r: (r, j)),
        ],
        out_specs=pl.BlockSpec((b, b), lambda i, j, r: (i, j)),
        out_shape=jax.ShapeDtypeStruct((np_, kp), jnp.bfloat16),
        scratch_shapes=[pltpu.VMEM((b, b), jnp.float32)],
    )(m, u)


def _mask_stage_body(ut_ref, w_ref, xa_ref, o_ref, acc_ref, *, nr, b):
    j = pl.program_id(0)
    i = pl.program_id(1)
    r = pl.program_id(2)

    @pl.when(r == 0)
    def _init():
        acc_ref[...] = jnp.zeros_like(acc_ref)

    acc_ref[...] += lax.dot_general(ut_ref[...], w_ref[...],
                                    (((0,), (0,)), ((), ())),
                                    preferred_element_type=jnp.float32)

    @pl.when(r == nr - 1)
    def _done():
        gi = i * b + lax.broadcasted_iota(jnp.int32, (b, b), 0)
        gj = j * b + lax.broadcasted_iota(jnp.int32, (b, b), 1)
        mask = jnp.where((acc_ref[...] > 0.0) & (gi != gj), 1.0, 0.0)
        contrib = lax.dot_general(mask, xa_ref[...],
                                  (((0,), (0,)), ((), ())),
                                  preferred_element_type=jnp.float32)

        @pl.when(i == 0)
        def _set():
            o_ref[...] = contrib

        @pl.when(i != 0)
        def _add():
            o_ref[...] += contrib


def _mask_stage(u, w, xa):
    np_, kp = u.shape
    f = xa.shape[1]
    b = _BLK
    nj, ni, nr = kp // b, kp // b, np_ // b
    return pl.pallas_call(
        functools.partial(_mask_stage_body, nr=nr, b=b),
        grid=(nj, ni, nr),
        in_specs=[
            pl.BlockSpec((b, b), lambda j, i, r: (r, i)),
            pl.BlockSpec((b, b), lambda j, i, r: (r, j)),
            pl.BlockSpec((b, f), lambda j, i, r: (i, 0)),
        ],
        out_specs=pl.BlockSpec((b, f), lambda j, i, r: (j, 0)),
        out_shape=jax.ShapeDtypeStruct((kp, f), jnp.float32),
        scratch_shapes=[pltpu.VMEM((b, b), jnp.float32)],
    )(u, w, xa)


def _final_reduce_body(wd_ref, xa_ref, o_ref, *, d):
    q = pl.program_id(0)

    @pl.when(q == 0)
    def _init():
        o_ref[...] = jnp.zeros_like(o_ref)

    w = wd_ref[:, :d]
    deg = wd_ref[:, d:d + 1]
    m = w / jnp.maximum(deg, 1.0)
    o_ref[0:1, :d] += jnp.sum(m, axis=0, keepdims=True)
    o_ref[1:2, :d] += jnp.sum(xa_ref[:, :d], axis=0, keepdims=True)


def _final_reduce(wd, xa, d):
    kp, f = wd.shape
    b = _BLK
    return pl.pallas_call(
        functools.partial(_final_reduce_body, d=d),
        grid=(kp // b,),
        in_specs=[
            pl.BlockSpec((b, f), lambda q: (q, 0)),
            pl.BlockSpec((b, f), lambda q: (q, 0)),
        ],
        out_specs=pl.BlockSpec((8, f), lambda q: (0, 0)),
        out_shape=jax.ShapeDtypeStruct((8, f), jnp.float32),
    )(wd, xa)


def kernel(x, edge_index, batch, W_l1, b_l1, W_r1, W_lin, b_lin, W_att, b_att,
           W_le1, b_le1, W_le2, W_le3, b_le3, W_l2, b_l2, W_r2):
    n, d = x.shape
    k = int(math.ceil(0.5 * n))
    src = edge_index[0]
    dst = edge_index[1]
    ones_e = jnp.ones(src.shape, jnp.float32)

    msum, cnt = _sc_gather_segsum(x, src, dst, None, None)
    mean1 = msum / jnp.clip(cnt, 1.0, None)[:, None]
    h = jax.nn.relu(mean1 @ W_l1.T + b_l1 + x @ W_r1.T)

    hs = h[src]
    x_q = jnp.maximum(jax.ops.segment_max(hs, dst, num_segments=n), h)
    wq = W_att[0, :d]
    wj = W_att[0, d:]
    alpha = x_q @ (W_lin.T @ wq) + (b_lin @ wq + b_att[0])
    beta = h @ wj
    ex_loop = _leaky_exp(alpha + beta)
    xp_num, den_e = _sc_gather_segsum(h, src, dst, alpha, beta)
    den = den_e + ex_loop
    xp = (xp_num + h * ex_loop[:, None]) / den[:, None]

    a_n = xp @ W_le1[0] + b_le1[0]
    b_n = xp @ W_le2[0]
    agg = (_sc_scalar_segsum(a_n, src, dst, n) + a_n
           - (cnt + 1.0) * b_n)
    fitness = jax.nn.sigmoid(agg + xp @ W_le3[0] + b_le3[0])

    _, idx = lax.top_k(fitness, k)
    sel = jnp.sort(idx)
    xsel = xp[sel] * fitness[sel][:, None]

    b_ = _BLK
    np_ = ((n + b_ - 1) // b_) * b_
    kp = ((k + b_ - 1) // b_) * b_
    loops = jnp.arange(n, dtype=src.dtype)
    row = jnp.concatenate([src, loops])
    col = jnp.concatenate([dst, loops])
    ones_t = jnp.ones(row.shape, jnp.float32)
    m_pat = jnp.zeros((np_, np_), jnp.float32).at[row, col].add(
        ones_t).astype(jnp.bfloat16)
    pos_of = jnp.full((n,), -1, jnp.int32).at[sel].set(
        jnp.arange(k, dtype=jnp.int32))
    pc = pos_of[col]
    posd = jnp.where(pc >= 0, pc, kp)
    u_pat = jnp.zeros((np_, kp + 128), jnp.float32).at[row, posd].add(
        ones_t)[:, :kp].astype(jnp.bfloat16)

    w_pat = _w_pattern(m_pat, u_pat)

    f = 2 * d
    xa = jnp.zeros((kp, f), jnp.float32)
    xa = xa.at[:k, :d].set(xsel)
    xa = xa.at[:k, d].set(1.0)
    wd = _mask_stage(u_pat, w_pat, xa)
    red = _final_reduce(wd, xa, d)
    mean_sum = red[0, :d]
    xsel_sum = red[1, :d]
    out = (mean_sum @ W_l2.T + k * b_l2 + xsel_sum @ W_r2.T) / k
    return out.reshape(1, d)
